# Initial kernel scaffold; baseline (speedup 1.0000x reference)
import jax
import jax.numpy as jnp
from jax.experimental import pallas as pl


def kernel(X_cat, X_num, tables, W1, b1, g1, be1, rm1, rv1, W2, b2, g2, be2, rm2, rv2, W3, b3):
    raise NotImplementedError("write your pallas kernel here")


# speedup vs baseline: 7.9762x; 7.9762x over previous
import functools

import jax
import jax.numpy as jnp
from jax import lax
from jax.experimental import pallas as pl
from jax.experimental.pallas import tpu as pltpu
from jax.experimental.pallas import tpu_sc as plsc

B = 16384
F = 26
V = 100000
D = 32
NUM = 13
N_IN = F * D + NUM
EPS = 1e-5

NW = 32
R = B * F
RPW = R // NW
IDX_ROWS = RPW // 128
SUB = 8
CH = SUB * 128
NCH = RPW // CH


def _gather_body(tab_hbm, xcat_hbm, offs_hbm, out_hbm, idx_v, offs_v, rows_v, gsem):
    wid = lax.axis_index("s") * 2 + lax.axis_index("c")
    base = wid * RPW

    pltpu.sync_copy(xcat_hbm.at[wid], idx_v)
    pltpu.sync_copy(offs_hbm.at[wid], offs_v)

    def fix_row(r, carry):
        for cc in range(8):
            sl = pl.ds(cc * 16, 16)
            idx_v[r, sl] = idx_v[r, sl] + offs_v[r, sl]
        return carry

    lax.fori_loop(0, IDX_ROWS, fix_row, 0)

    def chunk(c, carry):
        copies = []
        for jj in range(SUB):
            cp = pltpu.make_async_copy(
                tab_hbm.at[idx_v.at[c * SUB + jj]],
                rows_v.at[pl.ds(jj * 128, 128)],
                gsem,
            )
            cp.start()
            copies.append(cp)
        for cp in copies:
            cp.wait()
        pltpu.sync_copy(rows_v, out_hbm.at[pl.ds(base + c * CH, CH)])
        return carry

    lax.fori_loop(0, NCH, chunk, 0)


def _sc_gather(tab2, xcat, offs):
    mesh = plsc.VectorSubcoreMesh(core_axis_name="c", subcore_axis_name="s")
    k = pl.kernel(
        _gather_body,
        out_type=jax.ShapeDtypeStruct((R, D), jnp.float32),
        mesh=mesh,
        scratch_types=[
            pltpu.VMEM((IDX_ROWS, 128), jnp.int32),
            pltpu.VMEM((IDX_ROWS, 128), jnp.int32),
            pltpu.VMEM((CH, D), jnp.float32),
            pltpu.SemaphoreType.DMA,
        ],
        compiler_params=pltpu.CompilerParams(use_tc_tiling_on_sc=False),
    )
    return k(tab2, xcat, offs)


BT = 1024


def _mlp_body(cat_ref, num_ref, w1e_ref, w1n_ref, w2_ref, par_ref, out_ref):
    p = par_ref[...]
    b1 = p[0:1, :]
    a1 = p[1:2, :] * lax.rsqrt(p[4:5, :] + EPS)
    c1 = p[2:3, :] - p[3:4, :] * a1
    b2 = p[5:6, :]
    a2 = p[6:7, :] * lax.rsqrt(p[9:10, :] + EPS)
    c2 = p[7:8, :] - p[8:9, :] * a2
    w3 = p[10:11, :]
    b3 = p[11:12, 0:1]

    z = jnp.dot(cat_ref[...], w1e_ref[...], preferred_element_type=jnp.float32)
    z = z + jnp.dot(num_ref[...], w1n_ref[...], preferred_element_type=jnp.float32)
    h = jnp.maximum(z + b1, 0.0) * a1 + c1
    z2 = jnp.dot(h, w2_ref[...], preferred_element_type=jnp.float32)
    h2 = jnp.maximum(z2 + b2, 0.0) * a2 + c2
    z3 = jnp.sum(h2 * w3, axis=1, keepdims=True) + b3
    out_ref[...] = jax.nn.sigmoid(z3)


def _tc_mlp(cat2, xnum_p, w1e_t, w1n_t, w2_t, params):
    grid = (B // BT,)
    return pl.pallas_call(
        _mlp_body,
        grid=grid,
        in_specs=[
            pl.BlockSpec((BT, F * D), lambda i: (i, 0)),
            pl.BlockSpec((BT, 128), lambda i: (i, 0)),
            pl.BlockSpec((F * D, 64), lambda i: (0, 0)),
            pl.BlockSpec((128, 64), lambda i: (0, 0)),
            pl.BlockSpec((64, 64), lambda i: (0, 0)),
            pl.BlockSpec((16, 64), lambda i: (0, 0)),
        ],
        out_specs=pl.BlockSpec((BT, 1), lambda i: (i, 0)),
        out_shape=jax.ShapeDtypeStruct((B, 1), jnp.float32),
    )(cat2, xnum_p, w1e_t, w1n_t, w2_t, params)


def kernel(X_cat, X_num, tables, W1, b1, g1, be1, rm1, rv1,
           W2, b2, g2, be2, rm2, rv2, W3, b3):
    tab2 = tables.reshape(F * V, D)
    xcat = X_cat.reshape(NW, IDX_ROWS, 128)
    offs = jnp.tile(jnp.arange(F, dtype=jnp.int32) * V, B).reshape(NW, IDX_ROWS, 128)

    cat2 = _sc_gather(tab2, xcat, offs).reshape(B, F * D)

    xnum_p = jnp.pad(X_num, ((0, 0), (0, 128 - NUM)))
    w1e_t = W1[:, : F * D].T
    w1n_t = jnp.pad(W1[:, F * D:], ((0, 0), (0, 128 - NUM))).T
    w2_t = W2.T
    params = jnp.zeros((16, 64), jnp.float32)
    rows = [b1, g1, be1, rm1, rv1, b2, g2, be2, rm2, rv2,
            W3[0], jnp.full((64,), b3[0], jnp.float32)]
    params = params.at[: len(rows)].set(jnp.stack(rows))

    return _tc_mlp(cat2, xnum_p, w1e_t, w1n_t, w2_t, params)
